# Initial kernel scaffold; baseline (speedup 1.0000x reference)
import jax
import jax.numpy as jnp
from jax.experimental import pallas as pl


def kernel(inputs, hidden, edges, W_res, b_res, W1, b1, Wp, bp, W2, b2, W3, b3, Wo1, bo1, Wo2, bo2, Wo3, bo3):
    raise NotImplementedError("write your pallas kernel here")


# speedup vs baseline: 2.1040x; 2.1040x over previous
import functools

import jax
import jax.numpy as jnp
from jax.experimental import pallas as pl
from jax.experimental.pallas import tpu as pltpu

N = 64
HID = 64
IN = 6
E = N * (N - 1)
EPS = 1e-6


def _decoder_kernel(x_ref, wtf_ref,
                    A_ref, Bm_ref, C_ref, W1a_ref, W1b_ref, b1_ref,
                    Wp2_ref, Wp0_ref, bp_ref,
                    W2_ref, b2_ref, W3_ref, b3_ref,
                    Wres_ref, bres_ref,
                    Wo1_ref, bo1_ref, Wo2_ref, bo2_ref, Wo3_ref, bo3_ref,
                    out_ref, H_scr):
    x = x_ref[0]
    pos = x[:, 0:3]
    vel = x[:, 3:6]

    n1 = jnp.sqrt(jnp.sum(vel * vel, axis=1, keepdims=True))
    e1 = vel / (n1 + EPS)
    e1x, e1y, e1z = e1[:, 0:1], e1[:, 1:2], e1[:, 2:3]
    rx, ry, rz = 0.12, 0.35, 0.93
    c2x = e1y * rz - e1z * ry
    c2y = e1z * rx - e1x * rz
    c2z = e1x * ry - e1y * rx
    e2 = jnp.concatenate([c2x, c2y, c2z], axis=1)
    n2 = jnp.sqrt(jnp.sum(e2 * e2, axis=1, keepdims=True))
    e2 = e2 / (n2 + EPS)
    e2x, e2y, e2z = e2[:, 0:1], e2[:, 1:2], e2[:, 2:3]
    c3x = e1y * e2z - e1z * e2y
    c3y = e1z * e2x - e1x * e2z
    c3z = e1x * e2y - e1y * e2x
    e3 = jnp.concatenate([c3x, c3y, c3z], axis=1)

    relp = jnp.concatenate([
        jnp.sum(e1 * pos, axis=1, keepdims=True),
        jnp.sum(e2 * pos, axis=1, keepdims=True),
        jnp.sum(e3 * pos, axis=1, keepdims=True)], axis=1)
    relv = jnp.concatenate([
        jnp.sum(e1 * vel, axis=1, keepdims=True),
        jnp.sum(e2 * vel, axis=1, keepdims=True),
        jnp.sum(e3 * vel, axis=1, keepdims=True)], axis=1)
    rel_feat = jnp.concatenate([relp, relv], axis=1)

    E3 = jnp.stack([e1, e2, e3], axis=-1)
    E192 = E3.reshape(192, 3)
    dot = functools.partial(jnp.dot, preferred_element_type=jnp.float32)
    Upos = dot(E192, A_ref[:]).reshape(64, 3, HID)
    Uvel = dot(E192, Bm_ref[:]).reshape(64, 3, HID)
    U_all = jnp.concatenate([Upos, Uvel], axis=1)
    Up_all = dot(E192, Wp2_ref[:]).reshape(64, 3, HID)

    XC = dot(rel_feat, C_ref[:])
    const1 = (b1_ref[:] - dot(relp, W1a_ref[:])
              - dot(relv, W1b_ref[:])).reshape(64, 1, HID)
    constp = (bp_ref[:] - dot(relp, Wp0_ref[:])).reshape(64, 1, HID)

    for j in range(64):
        h1 = dot(x, U_all[j]) + XC + const1[j]
        g = jax.nn.sigmoid(dot(pos, Up_all[j]) + constp[j])
        hg = jnp.maximum(h1, 0.0) * g
        H_scr[j * 64:(j + 1) * 64, :] = hg

    H = H_scr[:]
    H2 = jnp.maximum(dot(H, W2_ref[:]) + b2_ref[:], 0.0)
    MSG = dot(H2, W3_ref[:]) + b3_ref[:]

    Mw = MSG * wtf_ref[0]
    agg = Mw.reshape(64, 64, HID).sum(axis=1) / 63.0

    aug = agg + dot(rel_feat, Wres_ref[:]) + bres_ref[:]
    hh = jnp.maximum(dot(aug, Wo1_ref[:]) + bo1_ref[:], 0.0)
    hh = jnp.maximum(dot(hh, Wo2_ref[:]) + bo2_ref[:], 0.0)
    pred = dot(hh, Wo3_ref[:]) + bo3_ref[:]

    p0, p1, p2 = pred[:, 0:1], pred[:, 1:2], pred[:, 2:3]
    v0, v1, v2 = pred[:, 3:4], pred[:, 4:5], pred[:, 5:6]
    og_p = e1 * p0 + e2 * p1 + e3 * p2
    og_v = e1 * v0 + e2 * v1 + e3 * v2
    out_ref[0] = x + jnp.concatenate([og_p, og_v], axis=1)


def kernel(inputs, hidden, edges, W_res, b_res, W1, b1, Wp, bp,
           W2, b2, W3, b3, Wo1, bo1, Wo2, bo2, Wo3, bo3):
    B = inputs.shape[0]

    w = edges[..., 1]
    t = w.reshape(B, 63, 64)
    t = jnp.concatenate([t, jnp.zeros((B, 63, 1), jnp.float32)], axis=2)
    grid_ij = jnp.concatenate(
        [jnp.zeros((B, 1), jnp.float32), t.reshape(B, 63 * 65)],
        axis=1).reshape(B, 64, 64)
    wtf = jnp.swapaxes(grid_ij, 1, 2).reshape(B, 64 * 64, 1)

    A = W1[0:3] + W1[9:12]
    Bm = W1[3:6] + W1[6:9]
    C = W1[12:18]
    W1a, W1b = W1[0:3], W1[3:6]
    Wp2 = Wp[0:3] + Wp[3:6]
    Wp0 = Wp[0:3]
    r1 = lambda v: v.reshape(1, -1)

    full = lambda s: pl.BlockSpec(s, lambda b: (0,) * len(s))
    out = pl.pallas_call(
        _decoder_kernel,
        grid=(B,),
        in_specs=[
            pl.BlockSpec((1, N, IN), lambda b: (b, 0, 0)),
            pl.BlockSpec((1, N * N, 1), lambda b: (b, 0, 0)),
            full((3, HID)), full((3, HID)), full((6, HID)),
            full((3, HID)), full((3, HID)), full((1, HID)),
            full((3, HID)), full((3, HID)), full((1, HID)),
            full((HID, HID)), full((1, HID)),
            full((HID, HID)), full((1, HID)),
            full((IN, HID)), full((1, HID)),
            full((HID, HID)), full((1, HID)),
            full((HID, HID)), full((1, HID)),
            full((HID, IN)), full((1, IN)),
        ],
        out_specs=pl.BlockSpec((1, N, IN), lambda b: (b, 0, 0)),
        out_shape=jax.ShapeDtypeStruct((B, N, IN), jnp.float32),
        scratch_shapes=[pltpu.VMEM((N * N, HID), jnp.float32)],
        compiler_params=pltpu.CompilerParams(
            dimension_semantics=("parallel",)),
    )(inputs, wtf,
      A, Bm, C, W1a, W1b, r1(b1), Wp2, Wp0, r1(bp),
      W2, r1(b2), W3, r1(b3), W_res, r1(b_res),
      Wo1, r1(bo1), Wo2, r1(bo2), Wo3, r1(bo3))
    return out


# speedup vs baseline: 2.6164x; 1.2435x over previous
import functools

import jax
import jax.numpy as jnp
from jax.experimental import pallas as pl
from jax.experimental.pallas import tpu as pltpu

N = 64
HID = 64
IN = 6
E = N * (N - 1)
EPS = 1e-6


def _decoder_kernel(x_ref, wt_ref,
                    A_ref, Bm_ref, C_ref, W1a_ref, W1b_ref, b1_ref,
                    Wp2_ref, Wp0_ref, bp_ref,
                    W2_ref, b2_ref, W3_ref, b3_ref,
                    Wres_ref, bres_ref,
                    Wo1_ref, bo1_ref, Wo2_ref, bo2_ref, Wo3_ref, bo3_ref,
                    out_ref, H_scr):
    x = x_ref[0]
    pos = x[:, 0:3]
    vel = x[:, 3:6]

    n1 = jnp.sqrt(jnp.sum(vel * vel, axis=1, keepdims=True))
    e1 = vel / (n1 + EPS)
    e1x, e1y, e1z = e1[:, 0:1], e1[:, 1:2], e1[:, 2:3]
    rx, ry, rz = 0.12, 0.35, 0.93
    c2x = e1y * rz - e1z * ry
    c2y = e1z * rx - e1x * rz
    c2z = e1x * ry - e1y * rx
    e2 = jnp.concatenate([c2x, c2y, c2z], axis=1)
    n2 = jnp.sqrt(jnp.sum(e2 * e2, axis=1, keepdims=True))
    e2 = e2 / (n2 + EPS)
    e2x, e2y, e2z = e2[:, 0:1], e2[:, 1:2], e2[:, 2:3]
    c3x = e1y * e2z - e1z * e2y
    c3y = e1z * e2x - e1x * e2z
    c3z = e1x * e2y - e1y * e2x
    e3 = jnp.concatenate([c3x, c3y, c3z], axis=1)

    relp = jnp.concatenate([
        jnp.sum(e1 * pos, axis=1, keepdims=True),
        jnp.sum(e2 * pos, axis=1, keepdims=True),
        jnp.sum(e3 * pos, axis=1, keepdims=True)], axis=1)
    relv = jnp.concatenate([
        jnp.sum(e1 * vel, axis=1, keepdims=True),
        jnp.sum(e2 * vel, axis=1, keepdims=True),
        jnp.sum(e3 * vel, axis=1, keepdims=True)], axis=1)
    rel_feat = jnp.concatenate([relp, relv], axis=1)

    E3 = jnp.stack([e1, e2, e3], axis=-1)
    E192 = E3.reshape(192, 3)
    dot = functools.partial(jnp.dot, preferred_element_type=jnp.float32)
    Upos = dot(E192, A_ref[:]).reshape(64, 3, HID)
    Uvel = dot(E192, Bm_ref[:]).reshape(64, 3, HID)
    U_all = jnp.concatenate([Upos, Uvel], axis=1)
    Up_all = dot(E192, Wp2_ref[:]).reshape(64, 3, HID)

    XC = dot(rel_feat, C_ref[:])
    const1 = (b1_ref[:] - dot(relp, W1a_ref[:])
              - dot(relv, W1b_ref[:])).reshape(64, 1, HID)
    constp = (bp_ref[:] - dot(relp, Wp0_ref[:])).reshape(64, 1, HID)

    for j in range(64):
        h1 = dot(x, U_all[j]) + XC + const1[j]
        g = jax.nn.sigmoid(dot(pos, Up_all[j]) + constp[j])
        hg = jnp.maximum(h1, 0.0) * g
        H_scr[j * 64:(j + 1) * 64, :] = hg

    H = H_scr[:]
    H2 = jnp.maximum(dot(H, W2_ref[:]) + b2_ref[:], 0.0)

    wt = wt_ref[0]
    rows = [dot(wt[j:j + 1, :], H2[j * 64:(j + 1) * 64, :])
            for j in range(64)]
    WH2 = jnp.concatenate(rows, axis=0)
    sw = jnp.sum(wt, axis=1, keepdims=True)
    agg = dot(WH2, W3_ref[:]) + sw * b3_ref[:]

    aug = agg + dot(rel_feat, Wres_ref[:]) + bres_ref[:]
    hh = jnp.maximum(dot(aug, Wo1_ref[:]) + bo1_ref[:], 0.0)
    hh = jnp.maximum(dot(hh, Wo2_ref[:]) + bo2_ref[:], 0.0)
    pred = dot(hh, Wo3_ref[:]) + bo3_ref[:]

    p0, p1, p2 = pred[:, 0:1], pred[:, 1:2], pred[:, 2:3]
    v0, v1, v2 = pred[:, 3:4], pred[:, 4:5], pred[:, 5:6]
    og_p = e1 * p0 + e2 * p1 + e3 * p2
    og_v = e1 * v0 + e2 * v1 + e3 * v2
    out_ref[0] = x + jnp.concatenate([og_p, og_v], axis=1)


def kernel(inputs, hidden, edges, W_res, b_res, W1, b1, Wp, bp,
           W2, b2, W3, b3, Wo1, bo1, Wo2, bo2, Wo3, bo3):
    B = inputs.shape[0]

    w = edges[..., 1]
    t = w.reshape(B, 63, 64)
    t = jnp.concatenate([t, jnp.zeros((B, 63, 1), jnp.float32)], axis=2)
    grid_ij = jnp.concatenate(
        [jnp.zeros((B, 1), jnp.float32), t.reshape(B, 63 * 65)],
        axis=1).reshape(B, 64, 64)
    wt = jnp.swapaxes(grid_ij, 1, 2) * (1.0 / 63.0)

    A = W1[0:3] + W1[9:12]
    Bm = W1[3:6] + W1[6:9]
    C = W1[12:18]
    W1a, W1b = W1[0:3], W1[3:6]
    Wp2 = Wp[0:3] + Wp[3:6]
    Wp0 = Wp[0:3]
    r1 = lambda v: v.reshape(1, -1)

    full = lambda s: pl.BlockSpec(s, lambda b: (0,) * len(s))
    out = pl.pallas_call(
        _decoder_kernel,
        grid=(B,),
        in_specs=[
            pl.BlockSpec((1, N, IN), lambda b: (b, 0, 0)),
            pl.BlockSpec((1, N, N), lambda b: (b, 0, 0)),
            full((3, HID)), full((3, HID)), full((6, HID)),
            full((3, HID)), full((3, HID)), full((1, HID)),
            full((3, HID)), full((3, HID)), full((1, HID)),
            full((HID, HID)), full((1, HID)),
            full((HID, HID)), full((1, HID)),
            full((IN, HID)), full((1, HID)),
            full((HID, HID)), full((1, HID)),
            full((HID, HID)), full((1, HID)),
            full((HID, IN)), full((1, IN)),
        ],
        out_specs=pl.BlockSpec((1, N, IN), lambda b: (b, 0, 0)),
        out_shape=jax.ShapeDtypeStruct((B, N, IN), jnp.float32),
        scratch_shapes=[pltpu.VMEM((N * N, HID), jnp.float32)],
        compiler_params=pltpu.CompilerParams(
            dimension_semantics=("parallel",)),
    )(inputs, wt,
      A, Bm, C, W1a, W1b, r1(b1), Wp2, Wp0, r1(bp),
      W2, r1(b2), W3, r1(b3), W_res, r1(b_res),
      Wo1, r1(bo1), Wo2, r1(bo2), Wo3, r1(bo3))
    return out


# speedup vs baseline: 2.6815x; 1.0249x over previous
import functools

import jax
import jax.numpy as jnp
from jax.experimental import pallas as pl
from jax.experimental.pallas import tpu as pltpu

N = 64
HID = 64
IN = 6
EPS = 1e-6


def _decoder_kernel(x_ref, wt_ref, WW_ref,
                    W2_ref, b2_ref, W3_ref, b3_ref,
                    Wres_ref, bres_ref,
                    Wo1_ref, bo1_ref, Wo2_ref, bo2_ref, Wo3_ref, bo3_ref,
                    out_ref, H_scr, agg_scr):
    x = x_ref[0]
    pos = x[:, 0:3]
    vel = x[:, 3:6]

    n1 = jnp.sqrt(jnp.sum(vel * vel, axis=1, keepdims=True))
    e1 = vel / (n1 + EPS)
    e1x, e1y, e1z = e1[:, 0:1], e1[:, 1:2], e1[:, 2:3]
    rx, ry, rz = 0.12, 0.35, 0.93
    c2x = e1y * rz - e1z * ry
    c2y = e1z * rx - e1x * rz
    c2z = e1x * ry - e1y * rx
    e2 = jnp.concatenate([c2x, c2y, c2z], axis=1)
    n2 = jnp.sqrt(jnp.sum(e2 * e2, axis=1, keepdims=True))
    e2 = e2 / (n2 + EPS)
    e2x, e2y, e2z = e2[:, 0:1], e2[:, 1:2], e2[:, 2:3]
    c3x = e1y * e2z - e1z * e2y
    c3y = e1z * e2x - e1x * e2z
    c3z = e1x * e2y - e1y * e2x
    e3 = jnp.concatenate([c3x, c3y, c3z], axis=1)

    relp = jnp.concatenate([
        jnp.sum(e1 * pos, axis=1, keepdims=True),
        jnp.sum(e2 * pos, axis=1, keepdims=True),
        jnp.sum(e3 * pos, axis=1, keepdims=True)], axis=1)
    relv = jnp.concatenate([
        jnp.sum(e1 * vel, axis=1, keepdims=True),
        jnp.sum(e2 * vel, axis=1, keepdims=True),
        jnp.sum(e3 * vel, axis=1, keepdims=True)], axis=1)
    rel_feat = jnp.concatenate([relp, relv], axis=1)

    dot = functools.partial(jnp.dot, preferred_element_type=jnp.float32)

    E3 = jnp.stack([e1, e2, e3], axis=-1)
    z = lambda *s: jnp.zeros(s, jnp.float32)
    E3p1 = jnp.concatenate([E3, z(64, 3, 16)], axis=2)
    E3p2 = jnp.concatenate([z(64, 3, 3), E3, z(64, 3, 13)], axis=2)
    eye = (jax.lax.broadcasted_iota(jnp.int32, (6, 19), 1)
           == jax.lax.broadcasted_iota(jnp.int32, (6, 19), 0) + 6)
    I6b = jnp.broadcast_to(eye.astype(jnp.float32)[None], (64, 6, 19))
    rf1 = rel_feat.reshape(64, 1, 6)
    crow = jnp.concatenate([z(64, 1, 12), rf1, jnp.ones((64, 1, 1))], axis=2)
    M3 = jnp.concatenate([E3p1, E3p2, I6b, crow, z(64, 3, 19)], axis=1)
    UW = dot(M3.reshape(1024, 19), WW_ref[:]).reshape(64, 16, 128)

    x16 = jnp.concatenate(
        [x, rel_feat, jnp.ones((64, 1), jnp.float32), z(64, 3)], axis=1)

    for j in range(64):
        t = dot(x16, UW[j])
        hg = jnp.maximum(t[:, :HID], 0.0) * jax.nn.sigmoid(t[:, HID:])
        H_scr[j * 64:(j + 1) * 64, :] = hg.astype(jnp.bfloat16)

    H2 = jnp.maximum(dot(H_scr[:], W2_ref[:]) + b2_ref[:], 0.0)

    wt = wt_ref[0]
    for j in range(64):
        agg_scr[j:j + 1, :] = dot(wt[j:j + 1, :], H2[j * 64:(j + 1) * 64, :])
    sw = jnp.sum(wt, axis=1, keepdims=True)
    agg = dot(agg_scr[:], W3_ref[:]) + sw * b3_ref[:]

    aug = agg + dot(rel_feat, Wres_ref[:]) + bres_ref[:]
    hh = jnp.maximum(dot(aug, Wo1_ref[:]) + bo1_ref[:], 0.0)
    hh = jnp.maximum(dot(hh, Wo2_ref[:]) + bo2_ref[:], 0.0)
    pred = dot(hh, Wo3_ref[:]) + bo3_ref[:]

    p0, p1, p2 = pred[:, 0:1], pred[:, 1:2], pred[:, 2:3]
    v0, v1, v2 = pred[:, 3:4], pred[:, 4:5], pred[:, 5:6]
    og_p = e1 * p0 + e2 * p1 + e3 * p2
    og_v = e1 * v0 + e2 * v1 + e3 * v2
    out_ref[0] = x + jnp.concatenate([og_p, og_v], axis=1)


def kernel(inputs, hidden, edges, W_res, b_res, W1, b1, Wp, bp,
           W2, b2, W3, b3, Wo1, bo1, Wo2, bo2, Wo3, bo3):
    B = inputs.shape[0]

    w = edges[..., 1]
    t = w.reshape(B, 63, 64)
    t = jnp.concatenate([t, jnp.zeros((B, 63, 1), jnp.float32)], axis=2)
    grid_ij = jnp.concatenate(
        [jnp.zeros((B, 1), jnp.float32), t.reshape(B, 63 * 65)],
        axis=1).reshape(B, 64, 64)
    wt = jnp.swapaxes(grid_ij, 1, 2) * (1.0 / 63.0)

    z364 = jnp.zeros((3, HID), jnp.float32)
    z664 = jnp.zeros((6, HID), jnp.float32)
    WW = jnp.concatenate([
        jnp.concatenate([W1[0:3] + W1[9:12], Wp[0:3] + Wp[3:6]], axis=1),
        jnp.concatenate([W1[3:6] + W1[6:9], z364], axis=1),
        jnp.concatenate([W1[12:18], z664], axis=1),
        -jnp.concatenate([W1[0:3], Wp[0:3]], axis=1),
        -jnp.concatenate([W1[3:6], z364], axis=1),
        jnp.concatenate([b1.reshape(1, -1), bp.reshape(1, -1)], axis=1),
    ], axis=0)
    r1 = lambda v: v.reshape(1, -1)

    full = lambda s: pl.BlockSpec(s, lambda b: (0,) * len(s))
    out = pl.pallas_call(
        _decoder_kernel,
        grid=(B,),
        in_specs=[
            pl.BlockSpec((1, N, IN), lambda b: (b, 0, 0)),
            pl.BlockSpec((1, N, N), lambda b: (b, 0, 0)),
            full((19, 2 * HID)),
            full((HID, HID)), full((1, HID)),
            full((HID, HID)), full((1, HID)),
            full((IN, HID)), full((1, HID)),
            full((HID, HID)), full((1, HID)),
            full((HID, HID)), full((1, HID)),
            full((HID, IN)), full((1, IN)),
        ],
        out_specs=pl.BlockSpec((1, N, IN), lambda b: (b, 0, 0)),
        out_shape=jax.ShapeDtypeStruct((B, N, IN), jnp.float32),
        scratch_shapes=[pltpu.VMEM((N * N, HID), jnp.bfloat16),
                        pltpu.VMEM((N, HID), jnp.float32)],
        compiler_params=pltpu.CompilerParams(
            dimension_semantics=("parallel",)),
    )(inputs, wt, WW,
      W2.astype(jnp.bfloat16), r1(b2), W3, r1(b3), W_res, r1(b_res),
      Wo1, r1(bo1), Wo2, r1(bo2), Wo3, r1(bo3))
    return out


# speedup vs baseline: 3.8711x; 1.4436x over previous
import functools

import jax
import jax.numpy as jnp
from jax.experimental import pallas as pl
from jax.experimental.pallas import tpu as pltpu

N = 64
HID = 64
IN = 6
BT = 4
EPS = 1e-6


def _decoder_kernel(x_ref, wt_ref, WW_ref,
                    W2_ref, b2_ref, W3_ref, b3_ref,
                    Wres_ref, bres_ref,
                    Wo1_ref, bo1_ref, Wo2_ref, bo2_ref, Wo3_ref, bo3_ref,
                    out_ref, H_scr, agg_scr):
    NB = BT * N
    x = x_ref[:].reshape(NB, IN)
    pos = x[:, 0:3]
    vel = x[:, 3:6]

    n1 = jnp.sqrt(jnp.sum(vel * vel, axis=1, keepdims=True))
    e1 = vel / (n1 + EPS)
    e1x, e1y, e1z = e1[:, 0:1], e1[:, 1:2], e1[:, 2:3]
    rx, ry, rz = 0.12, 0.35, 0.93
    c2x = e1y * rz - e1z * ry
    c2y = e1z * rx - e1x * rz
    c2z = e1x * ry - e1y * rx
    e2 = jnp.concatenate([c2x, c2y, c2z], axis=1)
    n2 = jnp.sqrt(jnp.sum(e2 * e2, axis=1, keepdims=True))
    e2 = e2 / (n2 + EPS)
    e2x, e2y, e2z = e2[:, 0:1], e2[:, 1:2], e2[:, 2:3]
    c3x = e1y * e2z - e1z * e2y
    c3y = e1z * e2x - e1x * e2z
    c3z = e1x * e2y - e1y * e2x
    e3 = jnp.concatenate([c3x, c3y, c3z], axis=1)

    relp = jnp.concatenate([
        jnp.sum(e1 * pos, axis=1, keepdims=True),
        jnp.sum(e2 * pos, axis=1, keepdims=True),
        jnp.sum(e3 * pos, axis=1, keepdims=True)], axis=1)
    relv = jnp.concatenate([
        jnp.sum(e1 * vel, axis=1, keepdims=True),
        jnp.sum(e2 * vel, axis=1, keepdims=True),
        jnp.sum(e3 * vel, axis=1, keepdims=True)], axis=1)
    rel_feat = jnp.concatenate([relp, relv], axis=1)

    dot = functools.partial(jnp.dot, preferred_element_type=jnp.float32)

    E3 = jnp.stack([e1, e2, e3], axis=-1)
    z = lambda *s: jnp.zeros(s, jnp.float32)
    E3p1 = jnp.concatenate([E3, z(NB, 3, 16)], axis=2)
    E3p2 = jnp.concatenate([z(NB, 3, 3), E3, z(NB, 3, 13)], axis=2)
    eye = (jax.lax.broadcasted_iota(jnp.int32, (6, 19), 1)
           == jax.lax.broadcasted_iota(jnp.int32, (6, 19), 0) + 6)
    I6b = jnp.broadcast_to(eye.astype(jnp.float32)[None], (NB, 6, 19))
    rf1 = rel_feat.reshape(NB, 1, 6)
    crow = jnp.concatenate([z(NB, 1, 12), rf1, jnp.ones((NB, 1, 1))], axis=2)
    M3 = jnp.concatenate([E3p1, E3p2, I6b, crow, z(NB, 3, 19)], axis=1)
    UW = dot(M3.reshape(NB * 16, 19), WW_ref[:]).reshape(NB, 16, 128)

    x16 = jnp.concatenate(
        [x, rel_feat, jnp.ones((NB, 1), jnp.float32), z(NB, 3)], axis=1)

    for b in range(BT):
        x16_b = x16[b * 64:(b + 1) * 64, :]
        for j in range(64):
            t = dot(x16_b, UW[b * 64 + j])
            hg = jnp.maximum(t[:, :HID], 0.0) * jax.nn.sigmoid(t[:, HID:])
            H_scr[(b * 64 + j) * 64:(b * 64 + j + 1) * 64, :] = (
                hg.astype(jnp.bfloat16))

    H2 = jnp.maximum(dot(H_scr[:], W2_ref[:]) + b2_ref[:], 0.0)

    sws = []
    for b in range(BT):
        wtb = wt_ref[b]
        sws.append(jnp.sum(wtb, axis=1, keepdims=True))
        for j in range(64):
            r = b * 64 + j
            agg_scr[r:r + 1, :] = dot(wtb[j:j + 1, :],
                                      H2[r * 64:(r + 1) * 64, :])
    sw = jnp.concatenate(sws, axis=0)
    agg = dot(agg_scr[:], W3_ref[:]) + sw * b3_ref[:]

    aug = agg + dot(rel_feat, Wres_ref[:]) + bres_ref[:]
    hh = jnp.maximum(dot(aug, Wo1_ref[:]) + bo1_ref[:], 0.0)
    hh = jnp.maximum(dot(hh, Wo2_ref[:]) + bo2_ref[:], 0.0)
    pred = dot(hh, Wo3_ref[:]) + bo3_ref[:]

    p0, p1, p2 = pred[:, 0:1], pred[:, 1:2], pred[:, 2:3]
    v0, v1, v2 = pred[:, 3:4], pred[:, 4:5], pred[:, 5:6]
    og_p = e1 * p0 + e2 * p1 + e3 * p2
    og_v = e1 * v0 + e2 * v1 + e3 * v2
    out_ref[:] = (x + jnp.concatenate([og_p, og_v], axis=1)).reshape(
        BT, N, IN)


def kernel(inputs, hidden, edges, W_res, b_res, W1, b1, Wp, bp,
           W2, b2, W3, b3, Wo1, bo1, Wo2, bo2, Wo3, bo3):
    B = inputs.shape[0]

    w = edges[..., 1]
    t = w.reshape(B, 63, 64)
    t = jnp.concatenate([t, jnp.zeros((B, 63, 1), jnp.float32)], axis=2)
    grid_ij = jnp.concatenate(
        [jnp.zeros((B, 1), jnp.float32), t.reshape(B, 63 * 65)],
        axis=1).reshape(B, 64, 64)
    wt = jnp.swapaxes(grid_ij, 1, 2) * (1.0 / 63.0)

    z364 = jnp.zeros((3, HID), jnp.float32)
    z664 = jnp.zeros((6, HID), jnp.float32)
    WW = jnp.concatenate([
        jnp.concatenate([W1[0:3] + W1[9:12], Wp[0:3] + Wp[3:6]], axis=1),
        jnp.concatenate([W1[3:6] + W1[6:9], z364], axis=1),
        jnp.concatenate([W1[12:18], z664], axis=1),
        -jnp.concatenate([W1[0:3], Wp[0:3]], axis=1),
        -jnp.concatenate([W1[3:6], z364], axis=1),
        jnp.concatenate([b1.reshape(1, -1), bp.reshape(1, -1)], axis=1),
    ], axis=0)
    r1 = lambda v: v.reshape(1, -1)

    full = lambda s: pl.BlockSpec(s, lambda b: (0,) * len(s))
    out = pl.pallas_call(
        _decoder_kernel,
        grid=(B // BT,),
        in_specs=[
            pl.BlockSpec((BT, N, IN), lambda b: (b, 0, 0)),
            pl.BlockSpec((BT, N, N), lambda b: (b, 0, 0)),
            full((19, 2 * HID)),
            full((HID, HID)), full((1, HID)),
            full((HID, HID)), full((1, HID)),
            full((IN, HID)), full((1, HID)),
            full((HID, HID)), full((1, HID)),
            full((HID, HID)), full((1, HID)),
            full((HID, IN)), full((1, IN)),
        ],
        out_specs=pl.BlockSpec((BT, N, IN), lambda b: (b, 0, 0)),
        out_shape=jax.ShapeDtypeStruct((B, N, IN), jnp.float32),
        scratch_shapes=[pltpu.VMEM((BT * N * N, HID), jnp.bfloat16),
                        pltpu.VMEM((BT * N, HID), jnp.float32)],
        compiler_params=pltpu.CompilerParams(
            dimension_semantics=("parallel",)),
    )(inputs, wt, WW,
      W2.astype(jnp.bfloat16), r1(b2), W3, r1(b3), W_res, r1(b_res),
      Wo1, r1(bo1), Wo2, r1(bo2), Wo3, r1(bo3))
    return out


# speedup vs baseline: 4.6276x; 1.1954x over previous
import functools

import jax
import jax.numpy as jnp
from jax.experimental import pallas as pl
from jax.experimental.pallas import tpu as pltpu

N = 64
HID = 64
IN = 6
BT = 8
EPS = 1e-6


def _decoder_kernel(x_ref, wt_ref, WW_ref,
                    W2_ref, b2_ref, W3_ref, b3_ref,
                    Wres_ref, bres_ref,
                    Wo1_ref, bo1_ref, Wo2_ref, bo2_ref, Wo3_ref, bo3_ref,
                    out_ref, H_scr, agg_scr):
    NB = BT * N
    x = x_ref[:].reshape(NB, IN)
    pos = x[:, 0:3]
    vel = x[:, 3:6]

    n1 = jnp.sqrt(jnp.sum(vel * vel, axis=1, keepdims=True))
    e1 = vel / (n1 + EPS)
    e1x, e1y, e1z = e1[:, 0:1], e1[:, 1:2], e1[:, 2:3]
    rx, ry, rz = 0.12, 0.35, 0.93
    c2x = e1y * rz - e1z * ry
    c2y = e1z * rx - e1x * rz
    c2z = e1x * ry - e1y * rx
    e2 = jnp.concatenate([c2x, c2y, c2z], axis=1)
    n2 = jnp.sqrt(jnp.sum(e2 * e2, axis=1, keepdims=True))
    e2 = e2 / (n2 + EPS)
    e2x, e2y, e2z = e2[:, 0:1], e2[:, 1:2], e2[:, 2:3]
    c3x = e1y * e2z - e1z * e2y
    c3y = e1z * e2x - e1x * e2z
    c3z = e1x * e2y - e1y * e2x
    e3 = jnp.concatenate([c3x, c3y, c3z], axis=1)

    relp = jnp.concatenate([
        jnp.sum(e1 * pos, axis=1, keepdims=True),
        jnp.sum(e2 * pos, axis=1, keepdims=True),
        jnp.sum(e3 * pos, axis=1, keepdims=True)], axis=1)
    relv = jnp.concatenate([
        jnp.sum(e1 * vel, axis=1, keepdims=True),
        jnp.sum(e2 * vel, axis=1, keepdims=True),
        jnp.sum(e3 * vel, axis=1, keepdims=True)], axis=1)
    rel_feat = jnp.concatenate([relp, relv], axis=1)

    dot = functools.partial(jnp.dot, preferred_element_type=jnp.float32)

    E3 = jnp.stack([e1, e2, e3], axis=-1)
    z = lambda *s: jnp.zeros(s, jnp.float32)
    E3p1 = jnp.concatenate([E3, z(NB, 3, 16)], axis=2)
    E3p2 = jnp.concatenate([z(NB, 3, 3), E3, z(NB, 3, 13)], axis=2)
    eye = (jax.lax.broadcasted_iota(jnp.int32, (6, 19), 1)
           == jax.lax.broadcasted_iota(jnp.int32, (6, 19), 0) + 6)
    I6b = jnp.broadcast_to(eye.astype(jnp.float32)[None], (NB, 6, 19))
    rf1 = rel_feat.reshape(NB, 1, 6)
    crow = jnp.concatenate([z(NB, 1, 12), rf1, jnp.ones((NB, 1, 1))], axis=2)
    M3 = jnp.concatenate([E3p1, E3p2, I6b, crow, z(NB, 3, 19)], axis=1)
    UW = dot(M3.reshape(NB * 16, 19), WW_ref[:]).reshape(NB, 16, 128)

    x16 = jnp.concatenate(
        [x, rel_feat, jnp.ones((NB, 1), jnp.float32), z(NB, 3)], axis=1)

    for b in range(BT):
        x16_b = x16[b * 64:(b + 1) * 64, :]
        for j in range(64):
            t = dot(x16_b, UW[b * 64 + j])
            hg = jnp.maximum(t[:, :HID], 0.0) * jax.nn.sigmoid(t[:, HID:])
            H_scr[(b * 64 + j) * 64:(b * 64 + j + 1) * 64, :] = (
                hg.astype(jnp.bfloat16))

    H2 = jnp.maximum(dot(H_scr[:], W2_ref[:]) + b2_ref[:], 0.0)

    sws = []
    for b in range(BT):
        wtb = wt_ref[b]
        sws.append(jnp.sum(wtb, axis=1, keepdims=True))
        for j in range(64):
            r = b * 64 + j
            agg_scr[r:r + 1, :] = dot(wtb[j:j + 1, :],
                                      H2[r * 64:(r + 1) * 64, :])
    sw = jnp.concatenate(sws, axis=0)
    agg = dot(agg_scr[:], W3_ref[:]) + sw * b3_ref[:]

    aug = agg + dot(rel_feat, Wres_ref[:]) + bres_ref[:]
    hh = jnp.maximum(dot(aug, Wo1_ref[:]) + bo1_ref[:], 0.0)
    hh = jnp.maximum(dot(hh, Wo2_ref[:]) + bo2_ref[:], 0.0)
    pred = dot(hh, Wo3_ref[:]) + bo3_ref[:]

    p0, p1, p2 = pred[:, 0:1], pred[:, 1:2], pred[:, 2:3]
    v0, v1, v2 = pred[:, 3:4], pred[:, 4:5], pred[:, 5:6]
    og_p = e1 * p0 + e2 * p1 + e3 * p2
    og_v = e1 * v0 + e2 * v1 + e3 * v2
    out_ref[:] = (x + jnp.concatenate([og_p, og_v], axis=1)).reshape(
        BT, N, IN)


def kernel(inputs, hidden, edges, W_res, b_res, W1, b1, Wp, bp,
           W2, b2, W3, b3, Wo1, bo1, Wo2, bo2, Wo3, bo3):
    B = inputs.shape[0]

    w = edges[..., 1]
    t = w.reshape(B, 63, 64)
    t = jnp.concatenate([t, jnp.zeros((B, 63, 1), jnp.float32)], axis=2)
    grid_ij = jnp.concatenate(
        [jnp.zeros((B, 1), jnp.float32), t.reshape(B, 63 * 65)],
        axis=1).reshape(B, 64, 64)
    wt = jnp.swapaxes(grid_ij, 1, 2) * (1.0 / 63.0)

    z364 = jnp.zeros((3, HID), jnp.float32)
    z664 = jnp.zeros((6, HID), jnp.float32)
    WW = jnp.concatenate([
        jnp.concatenate([W1[0:3] + W1[9:12], Wp[0:3] + Wp[3:6]], axis=1),
        jnp.concatenate([W1[3:6] + W1[6:9], z364], axis=1),
        jnp.concatenate([W1[12:18], z664], axis=1),
        -jnp.concatenate([W1[0:3], Wp[0:3]], axis=1),
        -jnp.concatenate([W1[3:6], z364], axis=1),
        jnp.concatenate([b1.reshape(1, -1), bp.reshape(1, -1)], axis=1),
    ], axis=0)
    r1 = lambda v: v.reshape(1, -1)

    full = lambda s: pl.BlockSpec(s, lambda b: (0,) * len(s))
    out = pl.pallas_call(
        _decoder_kernel,
        grid=(B // BT,),
        in_specs=[
            pl.BlockSpec((BT, N, IN), lambda b: (b, 0, 0)),
            pl.BlockSpec((BT, N, N), lambda b: (b, 0, 0)),
            full((19, 2 * HID)),
            full((HID, HID)), full((1, HID)),
            full((HID, HID)), full((1, HID)),
            full((IN, HID)), full((1, HID)),
            full((HID, HID)), full((1, HID)),
            full((HID, HID)), full((1, HID)),
            full((HID, IN)), full((1, IN)),
        ],
        out_specs=pl.BlockSpec((BT, N, IN), lambda b: (b, 0, 0)),
        out_shape=jax.ShapeDtypeStruct((B, N, IN), jnp.float32),
        scratch_shapes=[pltpu.VMEM((BT * N * N, HID), jnp.bfloat16),
                        pltpu.VMEM((BT * N, HID), jnp.float32)],
        compiler_params=pltpu.CompilerParams(
            dimension_semantics=("parallel",)),
    )(inputs, wt, WW,
      W2.astype(jnp.bfloat16), r1(b2), W3, r1(b3), W_res, r1(b_res),
      Wo1, r1(bo1), Wo2, r1(bo2), Wo3, r1(bo3))
    return out


# speedup vs baseline: 4.8304x; 1.0438x over previous
import functools

import jax
import jax.numpy as jnp
from jax.experimental import pallas as pl
from jax.experimental.pallas import tpu as pltpu

N = 64
HID = 64
IN = 6
BT = 8
EPS = 1e-6


def _decoder_kernel(x_ref, wt_ref, WW_ref,
                    W2_ref, b2_ref, W3_ref, b3_ref,
                    Wres_ref, bres_ref,
                    Wo1_ref, bo1_ref, Wo2_ref, bo2_ref, Wo3_ref, bo3_ref,
                    out_ref, H_scr, agg_scr):
    NB = BT * N
    x = x_ref[:].reshape(NB, IN)
    pos = x[:, 0:3]
    vel = x[:, 3:6]

    n1 = jnp.sqrt(jnp.sum(vel * vel, axis=1, keepdims=True))
    e1 = vel / (n1 + EPS)
    e1x, e1y, e1z = e1[:, 0:1], e1[:, 1:2], e1[:, 2:3]
    rx, ry, rz = 0.12, 0.35, 0.93
    c2x = e1y * rz - e1z * ry
    c2y = e1z * rx - e1x * rz
    c2z = e1x * ry - e1y * rx
    e2 = jnp.concatenate([c2x, c2y, c2z], axis=1)
    n2 = jnp.sqrt(jnp.sum(e2 * e2, axis=1, keepdims=True))
    e2 = e2 / (n2 + EPS)
    e2x, e2y, e2z = e2[:, 0:1], e2[:, 1:2], e2[:, 2:3]
    c3x = e1y * e2z - e1z * e2y
    c3y = e1z * e2x - e1x * e2z
    c3z = e1x * e2y - e1y * e2x
    e3 = jnp.concatenate([c3x, c3y, c3z], axis=1)

    relp = jnp.concatenate([
        jnp.sum(e1 * pos, axis=1, keepdims=True),
        jnp.sum(e2 * pos, axis=1, keepdims=True),
        jnp.sum(e3 * pos, axis=1, keepdims=True)], axis=1)
    relv = jnp.concatenate([
        jnp.sum(e1 * vel, axis=1, keepdims=True),
        jnp.sum(e2 * vel, axis=1, keepdims=True),
        jnp.sum(e3 * vel, axis=1, keepdims=True)], axis=1)
    rel_feat = jnp.concatenate([relp, relv], axis=1)

    dot = functools.partial(jnp.dot, preferred_element_type=jnp.float32)

    E3 = jnp.stack([e1, e2, e3], axis=-1)
    z = lambda *s: jnp.zeros(s, jnp.float32)
    E3p1 = jnp.concatenate([E3, z(NB, 3, 16)], axis=2)
    E3p2 = jnp.concatenate([z(NB, 3, 3), E3, z(NB, 3, 13)], axis=2)
    eye = (jax.lax.broadcasted_iota(jnp.int32, (6, 19), 1)
           == jax.lax.broadcasted_iota(jnp.int32, (6, 19), 0) + 6)
    I6b = jnp.broadcast_to(eye.astype(jnp.float32)[None], (NB, 6, 19))
    rf1 = rel_feat.reshape(NB, 1, 6)
    crow = jnp.concatenate([z(NB, 1, 12), rf1, jnp.ones((NB, 1, 1))], axis=2)
    M3 = jnp.concatenate([E3p1, E3p2, I6b, crow, z(NB, 3, 19)], axis=1)
    UW = dot(M3.reshape(NB * 16, 19), WW_ref[:]).reshape(NB, 16, 128)

    x16 = jnp.concatenate(
        [x, rel_feat, jnp.ones((NB, 1), jnp.float32), z(NB, 3)], axis=1)
    x16 = x16.astype(jnp.bfloat16)
    UWb = UW.astype(jnp.bfloat16)

    for b in range(BT):
        x16_b = x16[b * 64:(b + 1) * 64, :]
        for j in range(64):
            t = dot(x16_b, UWb[b * 64 + j])
            g = 0.5 * jnp.tanh(0.5 * t[:, HID:]) + 0.5
            hg = jnp.maximum(t[:, :HID], 0.0) * g
            H_scr[(b * 64 + j) * 64:(b * 64 + j + 1) * 64, :] = (
                hg.astype(jnp.bfloat16))

    H2 = jnp.maximum(dot(H_scr[:], W2_ref[:]) + b2_ref[:], 0.0)
    H2b = H2.astype(jnp.bfloat16)

    sws = []
    for b in range(BT):
        wtb = wt_ref[b]
        sws.append(jnp.sum(wtb, axis=1, keepdims=True))
        wtbb = wtb.astype(jnp.bfloat16)
        for j in range(64):
            r = b * 64 + j
            agg_scr[r:r + 1, :] = dot(wtbb[j:j + 1, :],
                                      H2b[r * 64:(r + 1) * 64, :])
    sw = jnp.concatenate(sws, axis=0)
    agg = dot(agg_scr[:], W3_ref[:]) + sw * b3_ref[:]

    aug = agg + dot(rel_feat, Wres_ref[:]) + bres_ref[:]
    hh = jnp.maximum(dot(aug, Wo1_ref[:]) + bo1_ref[:], 0.0)
    hh = jnp.maximum(dot(hh, Wo2_ref[:]) + bo2_ref[:], 0.0)
    pred = dot(hh, Wo3_ref[:]) + bo3_ref[:]

    p0, p1, p2 = pred[:, 0:1], pred[:, 1:2], pred[:, 2:3]
    v0, v1, v2 = pred[:, 3:4], pred[:, 4:5], pred[:, 5:6]
    og_p = e1 * p0 + e2 * p1 + e3 * p2
    og_v = e1 * v0 + e2 * v1 + e3 * v2
    out_ref[:] = (x + jnp.concatenate([og_p, og_v], axis=1)).reshape(
        BT, N, IN)


def kernel(inputs, hidden, edges, W_res, b_res, W1, b1, Wp, bp,
           W2, b2, W3, b3, Wo1, bo1, Wo2, bo2, Wo3, bo3):
    B = inputs.shape[0]

    w = edges[..., 1]
    t = w.reshape(B, 63, 64)
    t = jnp.concatenate([t, jnp.zeros((B, 63, 1), jnp.float32)], axis=2)
    grid_ij = jnp.concatenate(
        [jnp.zeros((B, 1), jnp.float32), t.reshape(B, 63 * 65)],
        axis=1).reshape(B, 64, 64)
    wt = jnp.swapaxes(grid_ij, 1, 2) * (1.0 / 63.0)

    z364 = jnp.zeros((3, HID), jnp.float32)
    z664 = jnp.zeros((6, HID), jnp.float32)
    WW = jnp.concatenate([
        jnp.concatenate([W1[0:3] + W1[9:12], Wp[0:3] + Wp[3:6]], axis=1),
        jnp.concatenate([W1[3:6] + W1[6:9], z364], axis=1),
        jnp.concatenate([W1[12:18], z664], axis=1),
        -jnp.concatenate([W1[0:3], Wp[0:3]], axis=1),
        -jnp.concatenate([W1[3:6], z364], axis=1),
        jnp.concatenate([b1.reshape(1, -1), bp.reshape(1, -1)], axis=1),
    ], axis=0)
    r1 = lambda v: v.reshape(1, -1)

    full = lambda s: pl.BlockSpec(s, lambda b: (0,) * len(s))
    out = pl.pallas_call(
        _decoder_kernel,
        grid=(B // BT,),
        in_specs=[
            pl.BlockSpec((BT, N, IN), lambda b: (b, 0, 0)),
            pl.BlockSpec((BT, N, N), lambda b: (b, 0, 0)),
            full((19, 2 * HID)),
            full((HID, HID)), full((1, HID)),
            full((HID, HID)), full((1, HID)),
            full((IN, HID)), full((1, HID)),
            full((HID, HID)), full((1, HID)),
            full((HID, HID)), full((1, HID)),
            full((HID, IN)), full((1, IN)),
        ],
        out_specs=pl.BlockSpec((BT, N, IN), lambda b: (b, 0, 0)),
        out_shape=jax.ShapeDtypeStruct((B, N, IN), jnp.float32),
        scratch_shapes=[pltpu.VMEM((BT * N * N, HID), jnp.bfloat16),
                        pltpu.VMEM((BT * N, HID), jnp.float32)],
        compiler_params=pltpu.CompilerParams(
            dimension_semantics=("parallel",)),
    )(inputs, wt, WW,
      W2.astype(jnp.bfloat16), r1(b2), W3, r1(b3), W_res, r1(b_res),
      Wo1, r1(bo1), Wo2, r1(bo2), Wo3, r1(bo3))
    return out


# speedup vs baseline: 4.8755x; 1.0093x over previous
import functools

import jax
import jax.numpy as jnp
from jax.experimental import pallas as pl
from jax.experimental.pallas import tpu as pltpu

N = 64
HID = 64
IN = 6
BT = 16
EPS = 1e-6


def _decoder_kernel(x_ref, wt_ref, WW_ref,
                    W2_ref, b2_ref, W3_ref, b3_ref,
                    Wres_ref, bres_ref,
                    Wo1_ref, bo1_ref, Wo2_ref, bo2_ref, Wo3_ref, bo3_ref,
                    out_ref, H_scr, agg_scr):
    NB = BT * N
    x = x_ref[:].reshape(NB, IN)
    pos = x[:, 0:3]
    vel = x[:, 3:6]

    n1 = jnp.sqrt(jnp.sum(vel * vel, axis=1, keepdims=True))
    e1 = vel / (n1 + EPS)
    e1x, e1y, e1z = e1[:, 0:1], e1[:, 1:2], e1[:, 2:3]
    rx, ry, rz = 0.12, 0.35, 0.93
    c2x = e1y * rz - e1z * ry
    c2y = e1z * rx - e1x * rz
    c2z = e1x * ry - e1y * rx
    e2 = jnp.concatenate([c2x, c2y, c2z], axis=1)
    n2 = jnp.sqrt(jnp.sum(e2 * e2, axis=1, keepdims=True))
    e2 = e2 / (n2 + EPS)
    e2x, e2y, e2z = e2[:, 0:1], e2[:, 1:2], e2[:, 2:3]
    c3x = e1y * e2z - e1z * e2y
    c3y = e1z * e2x - e1x * e2z
    c3z = e1x * e2y - e1y * e2x
    e3 = jnp.concatenate([c3x, c3y, c3z], axis=1)

    relp = jnp.concatenate([
        jnp.sum(e1 * pos, axis=1, keepdims=True),
        jnp.sum(e2 * pos, axis=1, keepdims=True),
        jnp.sum(e3 * pos, axis=1, keepdims=True)], axis=1)
    relv = jnp.concatenate([
        jnp.sum(e1 * vel, axis=1, keepdims=True),
        jnp.sum(e2 * vel, axis=1, keepdims=True),
        jnp.sum(e3 * vel, axis=1, keepdims=True)], axis=1)
    rel_feat = jnp.concatenate([relp, relv], axis=1)

    dot = functools.partial(jnp.dot, preferred_element_type=jnp.float32)

    E3 = jnp.stack([e1, e2, e3], axis=-1)
    z = lambda *s: jnp.zeros(s, jnp.float32)
    E3p1 = jnp.concatenate([E3, z(NB, 3, 16)], axis=2)
    E3p2 = jnp.concatenate([z(NB, 3, 3), E3, z(NB, 3, 13)], axis=2)
    eye = (jax.lax.broadcasted_iota(jnp.int32, (6, 19), 1)
           == jax.lax.broadcasted_iota(jnp.int32, (6, 19), 0) + 6)
    I6b = jnp.broadcast_to(eye.astype(jnp.float32)[None], (NB, 6, 19))
    rf1 = rel_feat.reshape(NB, 1, 6)
    crow = jnp.concatenate([z(NB, 1, 12), rf1, jnp.ones((NB, 1, 1))], axis=2)
    M3 = jnp.concatenate([E3p1, E3p2, I6b, crow, z(NB, 3, 19)], axis=1)
    UW = dot(M3.reshape(NB * 16, 19), WW_ref[:]).reshape(NB, 16, 128)

    x16 = jnp.concatenate(
        [x, rel_feat, jnp.ones((NB, 1), jnp.float32), z(NB, 3)], axis=1)
    x16 = x16.astype(jnp.bfloat16)
    UWb = UW.astype(jnp.bfloat16)

    for b in range(BT):
        x16_b = x16[b * 64:(b + 1) * 64, :]
        for j in range(64):
            t = dot(x16_b, UWb[b * 64 + j])
            g = 0.5 * jnp.tanh(0.5 * t[:, HID:]) + 0.5
            hg = jnp.maximum(t[:, :HID], 0.0) * g
            H_scr[(b * 64 + j) * 64:(b * 64 + j + 1) * 64, :] = (
                hg.astype(jnp.bfloat16))

    sws = []
    for b in range(BT):
        H2b = jnp.maximum(
            dot(H_scr[b * 4096:(b + 1) * 4096, :], W2_ref[:]) + b2_ref[:],
            0.0).astype(jnp.bfloat16)
        wtb = wt_ref[b]
        sws.append(jnp.sum(wtb, axis=1, keepdims=True))
        wtbb = wtb.astype(jnp.bfloat16)
        for j in range(64):
            agg_scr[b * 64 + j:b * 64 + j + 1, :] = dot(
                wtbb[j:j + 1, :], H2b[j * 64:(j + 1) * 64, :])
    sw = jnp.concatenate(sws, axis=0)
    agg = dot(agg_scr[:], W3_ref[:]) + sw * b3_ref[:]

    aug = agg + dot(rel_feat, Wres_ref[:]) + bres_ref[:]
    hh = jnp.maximum(dot(aug, Wo1_ref[:]) + bo1_ref[:], 0.0)
    hh = jnp.maximum(dot(hh, Wo2_ref[:]) + bo2_ref[:], 0.0)
    pred = dot(hh, Wo3_ref[:]) + bo3_ref[:]

    p0, p1, p2 = pred[:, 0:1], pred[:, 1:2], pred[:, 2:3]
    v0, v1, v2 = pred[:, 3:4], pred[:, 4:5], pred[:, 5:6]
    og_p = e1 * p0 + e2 * p1 + e3 * p2
    og_v = e1 * v0 + e2 * v1 + e3 * v2
    out_ref[:] = (x + jnp.concatenate([og_p, og_v], axis=1)).reshape(
        BT, N, IN)


def kernel(inputs, hidden, edges, W_res, b_res, W1, b1, Wp, bp,
           W2, b2, W3, b3, Wo1, bo1, Wo2, bo2, Wo3, bo3):
    B = inputs.shape[0]

    w = edges[..., 1]
    t = w.reshape(B, 63, 64)
    t = jnp.concatenate([t, jnp.zeros((B, 63, 1), jnp.float32)], axis=2)
    grid_ij = jnp.concatenate(
        [jnp.zeros((B, 1), jnp.float32), t.reshape(B, 63 * 65)],
        axis=1).reshape(B, 64, 64)
    wt = jnp.swapaxes(grid_ij, 1, 2) * (1.0 / 63.0)

    z364 = jnp.zeros((3, HID), jnp.float32)
    z664 = jnp.zeros((6, HID), jnp.float32)
    WW = jnp.concatenate([
        jnp.concatenate([W1[0:3] + W1[9:12], Wp[0:3] + Wp[3:6]], axis=1),
        jnp.concatenate([W1[3:6] + W1[6:9], z364], axis=1),
        jnp.concatenate([W1[12:18], z664], axis=1),
        -jnp.concatenate([W1[0:3], Wp[0:3]], axis=1),
        -jnp.concatenate([W1[3:6], z364], axis=1),
        jnp.concatenate([b1.reshape(1, -1), bp.reshape(1, -1)], axis=1),
    ], axis=0)
    r1 = lambda v: v.reshape(1, -1)

    full = lambda s: pl.BlockSpec(s, lambda b: (0,) * len(s))
    out = pl.pallas_call(
        _decoder_kernel,
        grid=(B // BT,),
        in_specs=[
            pl.BlockSpec((BT, N, IN), lambda b: (b, 0, 0)),
            pl.BlockSpec((BT, N, N), lambda b: (b, 0, 0)),
            full((19, 2 * HID)),
            full((HID, HID)), full((1, HID)),
            full((HID, HID)), full((1, HID)),
            full((IN, HID)), full((1, HID)),
            full((HID, HID)), full((1, HID)),
            full((HID, HID)), full((1, HID)),
            full((HID, IN)), full((1, IN)),
        ],
        out_specs=pl.BlockSpec((BT, N, IN), lambda b: (b, 0, 0)),
        out_shape=jax.ShapeDtypeStruct((B, N, IN), jnp.float32),
        scratch_shapes=[pltpu.VMEM((BT * N * N, HID), jnp.bfloat16),
                        pltpu.VMEM((BT * N, HID), jnp.float32)],
        compiler_params=pltpu.CompilerParams(
            dimension_semantics=("parallel",)),
    )(inputs, wt, WW,
      W2.astype(jnp.bfloat16), r1(b2), W3, r1(b3), W_res, r1(b_res),
      Wo1, r1(bo1), Wo2, r1(bo2), Wo3, r1(bo3))
    return out
